# speedup vs baseline: 3.1528x; 3.1528x over previous
import functools

import jax
import jax.numpy as jnp
from jax import lax
from jax.experimental import pallas as pl
from jax.experimental.pallas import tpu as pltpu
from jax.experimental.pallas import tpu_sc as plsc

VOCAB = 10000
DIM = 128
N_OUT = 128
B = 4096
L = 50

NC = 2
NS = 16
NW = NC * NS
TOKENS = B * L
B_PER_W = TOKENS // NW
CHUNK = 128
NCH = B_PER_W // CHUNK
NBUF = 5

ROWS_BLK = 1000


def _fold_body(emb_ref, w_ref, b_ref, out_ref):
    out_ref[...] = lax.dot_general(
        emb_ref[...], w_ref[...],
        dimension_numbers=(((1,), (1,)), ((), ())),
        preferred_element_type=jnp.float32,
    ) + b_ref[...]


def _fold_table(emb, W, b2):
    return pl.pallas_call(
        _fold_body,
        grid=(VOCAB // ROWS_BLK,),
        in_specs=[
            pl.BlockSpec((ROWS_BLK, DIM), lambda i: (i, 0)),
            pl.BlockSpec((N_OUT, DIM), lambda i: (0, 0)),
            pl.BlockSpec((1, N_OUT), lambda i: (0, 0)),
        ],
        out_specs=pl.BlockSpec((ROWS_BLK, N_OUT), lambda i: (i, 0)),
        out_shape=jax.ShapeDtypeStruct((VOCAB, N_OUT), jnp.float32),
    )(emb, W, b2)


def _sc_body(t_hbm, x_hbm, out_hbm, idx_v, rows_v, *sems):
    wid = lax.axis_index("s") * NC + lax.axis_index("c")
    base = wid * B_PER_W

    pltpu.sync_copy(x_hbm.at[wid], idx_v)

    def start_gather(c, buf):
        pltpu.async_copy(t_hbm.at[idx_v.at[c]], rows_v.at[buf], sems[buf])

    def wait_gather(c, buf):
        pltpu.make_async_copy(
            t_hbm.at[idx_v.at[c]], rows_v.at[buf], sems[buf]).wait()

    def start_out(c, buf):
        pltpu.async_copy(
            rows_v.at[buf], out_hbm.at[pl.ds(base + c * CHUNK, CHUNK)],
            sems[buf])

    def wait_out(c, buf):
        pltpu.make_async_copy(
            rows_v.at[buf], out_hbm.at[pl.ds(base + c * CHUNK, CHUNK)],
            sems[buf]).wait()

    for b in range(NBUF):
        start_gather(b, b)

    @pl.loop(0, NCH - NBUF, step=NBUF)
    def _group(g):
        for b in range(NBUF):
            c = g + b
            wait_gather(c, b)
            start_out(c, b)
            wait_out(c, b)
            start_gather(c + NBUF, b)

    for b in range(NBUF):
        c = NCH - NBUF + b
        wait_gather(c, b)
        start_out(c, b)
        wait_out(c, b)


def _sc_gather(T, x3):
    mesh = plsc.VectorSubcoreMesh(
        core_axis_name="c", subcore_axis_name="s", num_cores=NC,
        num_subcores=NS)
    run = pl.kernel(
        _sc_body,
        out_type=jax.ShapeDtypeStruct((TOKENS, N_OUT), jnp.float32),
        mesh=mesh,
        scratch_types=[
            pltpu.VMEM((NCH, CHUNK), jnp.int32),
            pltpu.VMEM((NBUF, CHUNK, N_OUT), jnp.float32),
        ] + [pltpu.SemaphoreType.DMA] * NBUF,
    )
    return run(T, x3)


@jax.jit
def kernel(x, emb, W, b):
    T = _fold_table(emb, W, b.reshape(1, N_OUT))
    x3 = x.astype(jnp.int32).reshape(NW, NCH, CHUNK)
    out = _sc_gather(T, x3)
    return out.reshape(B, L, N_OUT)


# speedup vs baseline: 5.3939x; 1.7108x over previous
import functools

import jax
import jax.numpy as jnp
from jax import lax
from jax.experimental import pallas as pl
from jax.experimental.pallas import tpu as pltpu
from jax.experimental.pallas import tpu_sc as plsc

VOCAB = 10000
DIM = 128
N_OUT = 128
B = 4096
L = 50

NC = 2
NS = 16
NW = NC * NS
TOKENS = B * L
CHUNK = L
NCH = B // NW
NBUF = 4

ROWS_BLK = 1000


def _fold_body(emb_ref, w_ref, b_ref, out_ref):
    out_ref[...] = lax.dot_general(
        emb_ref[...], w_ref[...],
        dimension_numbers=(((1,), (1,)), ((), ())),
        preferred_element_type=jnp.float32,
    ) + b_ref[...]


def _fold_table(emb, W, b2):
    return pl.pallas_call(
        _fold_body,
        grid=(VOCAB // ROWS_BLK,),
        in_specs=[
            pl.BlockSpec((ROWS_BLK, DIM), lambda i: (i, 0)),
            pl.BlockSpec((N_OUT, DIM), lambda i: (0, 0)),
            pl.BlockSpec((1, N_OUT), lambda i: (0, 0)),
        ],
        out_specs=pl.BlockSpec((ROWS_BLK, N_OUT), lambda i: (i, 0)),
        out_shape=jax.ShapeDtypeStruct((VOCAB, N_OUT), jnp.float32),
    )(emb, W, b2)


def _sc_body(t_hbm, x_hbm, out_hbm, idx_v, rows_v, *sems):
    wid = lax.axis_index("s") * NC + lax.axis_index("c")
    base = wid * NCH

    pltpu.sync_copy(x_hbm.at[wid], idx_v)

    def start_gather(c, buf):
        pltpu.async_copy(t_hbm.at[idx_v.at[c]], rows_v.at[buf], sems[buf])

    def wait_gather(c, buf):
        pltpu.make_async_copy(
            t_hbm.at[idx_v.at[c]], rows_v.at[buf], sems[buf]).wait()

    def start_out(c, buf):
        pltpu.async_copy(rows_v.at[buf], out_hbm.at[base + c], sems[buf])

    def wait_out(c, buf):
        pltpu.make_async_copy(
            rows_v.at[buf], out_hbm.at[base + c], sems[buf]).wait()

    for b in range(NBUF):
        start_gather(b, b)

    @pl.loop(0, NCH - NBUF, step=NBUF)
    def _group(g):
        for b in range(NBUF):
            c = g + b
            wait_gather(c, b)
            start_out(c, b)
            wait_out(c, b)
            start_gather(c + NBUF, b)

    for b in range(NBUF):
        c = NCH - NBUF + b
        wait_gather(c, b)
        start_out(c, b)
        wait_out(c, b)


def _sc_gather(T, x3):
    mesh = plsc.VectorSubcoreMesh(
        core_axis_name="c", subcore_axis_name="s", num_cores=NC,
        num_subcores=NS)
    run = pl.kernel(
        _sc_body,
        out_type=jax.ShapeDtypeStruct((B, L, N_OUT), jnp.float32),
        mesh=mesh,
        scratch_types=[
            pltpu.VMEM((NCH, CHUNK), jnp.int32),
            pltpu.VMEM((NBUF, CHUNK, N_OUT), jnp.float32),
        ] + [pltpu.SemaphoreType.DMA] * NBUF,
    )
    return run(T, x3)


@jax.jit
def kernel(x, emb, W, b):
    T = _fold_table(emb, W, b.reshape(1, N_OUT))
    x3 = x.astype(jnp.int32).reshape(NW, NCH, CHUNK)
    return _sc_gather(T, x3)
